# Initial kernel scaffold; baseline (speedup 1.0000x reference)
import jax
import jax.numpy as jnp
from jax.experimental import pallas as pl


def kernel(indices, table):
    raise NotImplementedError("write your pallas kernel here")


# speedup vs baseline: 4.8066x; 4.8066x over previous
import functools

import jax
import jax.numpy as jnp
import numpy as np
from jax import lax
from jax.experimental import pallas as pl
from jax.experimental.pallas import tpu as pltpu
from jax.experimental.pallas import tpu_sc as plsc

BATCH = 16384
HIST = 200
EMBED_DIM = 6
NTOK = BATCH * HIST
NC, NS = 2, 16
NW = NC * NS
TPW = NTOK // NW
CHUNK = 2048
NCHUNK = TPW // CHUNK
GROUP = 16
L = 16

_lane = np.arange(L * EMBED_DIM, dtype=np.int32)
_PATS = np.concatenate([
    (_lane // EMBED_DIM).reshape(EMBED_DIM, L),
    (_lane % EMBED_DIM).reshape(EMBED_DIM, L),
], axis=0)


def _permute(vec, idx):
    return lax.gather(
        vec, idx[:, None],
        dimension_numbers=lax.GatherDimensionNumbers(
            offset_dims=(), collapsed_slice_dims=(0,), start_index_map=(0,)),
        slice_sizes=(1,),
        mode=lax.GatherScatterMode.PROMISE_IN_BOUNDS)


def _sc_lookup(idx_flat, tab_pad, pats):
    mesh = plsc.VectorSubcoreMesh(core_axis_name="c", subcore_axis_name="s")

    @functools.partial(
        pl.kernel,
        mesh=mesh,
        out_type=jax.ShapeDtypeStruct((NTOK * EMBED_DIM,), jnp.float32),
        scratch_types=[
            pltpu.VMEM((CHUNK,), jnp.int32),
            pltpu.VMEM((CHUNK * EMBED_DIM,), jnp.float32),
            pltpu.VMEM((2 * L,), jnp.float32),
            pltpu.VMEM(_PATS.shape, jnp.int32),
        ],
    )
    def k(idx_hbm, tab_hbm, pats_hbm, out_hbm, idx_v, out_v, tab_v, pats_v):
        wid = lax.axis_index("s") * NC + lax.axis_index("c")
        wbase = wid * TPW

        pltpu.sync_copy(tab_hbm, tab_v)
        pltpu.sync_copy(pats_hbm, pats_v)

        t0 = tab_v[pl.ds(0, L)]
        t1 = tab_v[pl.ds(L, L)]
        gpat, w0, dw = [], [], []
        for v in range(EMBED_DIM):
            gpat.append(pats_v[v, :])
            cp = pats_v[EMBED_DIM + v, :]
            r0 = _permute(t0, cp)
            r1 = _permute(t1, cp)
            w0.append(r0)
            dw.append(r1 - r0)

        def group_body(g, carry):
            iv = idx_v[pl.ds(g * GROUP, GROUP)]
            for v in range(EMBED_DIM):
                pv = _permute(iv, gpat[v])
                ov = w0[v] + pv.astype(jnp.float32) * dw[v]
                out_v[pl.ds(g * (GROUP * EMBED_DIM) + v * L, L)] = ov
            return carry

        def chunk_body(ch, carry):
            ibase = wbase + ch * CHUNK
            pltpu.sync_copy(idx_hbm.at[pl.ds(ibase, CHUNK)], idx_v)
            lax.fori_loop(0, CHUNK // GROUP, group_body, 0)
            pltpu.sync_copy(out_v, out_hbm.at[pl.ds(ibase * EMBED_DIM,
                                                    CHUNK * EMBED_DIM)])
            return carry

        lax.fori_loop(0, NCHUNK, chunk_body, 0)

    return k(idx_flat, tab_pad, pats)


def kernel(indices, table):
    idx_flat = indices.reshape(-1).astype(jnp.int32)
    tab_pad = jnp.zeros((2, L), jnp.float32).at[:, :EMBED_DIM].set(table).reshape(-1)
    pats = jnp.asarray(_PATS)
    out_flat = _sc_lookup(idx_flat, tab_pad, pats)
    return out_flat.reshape(BATCH, HIST, EMBED_DIM)
